# baseline (speedup 1.0000x reference)
import jax
import jax.numpy as jnp
import numpy as np
from jax.experimental import pallas as pl
from jax.experimental.pallas import tpu as pltpu

_N_EXPERTS = 8
_N_ACTIVE = 5
_TEMP = float(np.e)
_TB = 1024
_SUB = 256


def _moe_body(x_ref, gw_ref, gb_ref, w1_ref, b1_ref, w2_ref, b2_ref,
              w3_ref, b3_ref, out_ref, wts_ref):
    e = pl.program_id(1)

    @pl.when(e == 0)
    def _gate():
        scores = (jnp.dot(x_ref[...], gw_ref[...],
                          preferred_element_type=jnp.float32)
                  + gb_ref[...]) / _TEMP
        m = jnp.max(scores, axis=-1, keepdims=True)
        ex = jnp.exp(scores - m)
        probs = ex / jnp.sum(ex, axis=-1, keepdims=True)
        idx = jax.lax.broadcasted_iota(jnp.int32, probs.shape, 1)
        cols = []
        for i in range(_N_EXPERTS):
            pi = probs[:, i:i + 1]
            beats = (probs > pi).astype(jnp.float32) + \
                jnp.where(probs == pi, (idx < i).astype(jnp.float32), 0.0)
            rank = jnp.sum(beats, axis=-1, keepdims=True)
            cols.append((rank < _N_ACTIVE).astype(jnp.float32))
        mask = jnp.concatenate(cols, axis=-1)
        w = probs * mask
        wts_ref[...] = w / (jnp.sum(w, axis=-1, keepdims=True) + 1e-8)

    w1b = w1_ref[0].astype(jnp.bfloat16)
    w2b = w2_ref[0].astype(jnp.bfloat16)
    w3b = w3_ref[0].astype(jnp.bfloat16)
    onehot = (jax.lax.broadcasted_iota(jnp.int32, (_SUB, _N_EXPERTS), 1)
              == e).astype(jnp.float32)

    for s in range(_TB // _SUB):
        sl = pl.ds(s * _SUB, _SUB)
        xs = x_ref[sl, :].astype(jnp.bfloat16)
        h1 = jnp.maximum(
            jnp.dot(xs, w1b, preferred_element_type=jnp.float32)
            + b1_ref[0], 0.0).astype(jnp.bfloat16)
        h2 = jnp.maximum(
            jnp.dot(h1, w2b, preferred_element_type=jnp.float32)
            + b2_ref[0], 0.0).astype(jnp.bfloat16)
        o = jnp.dot(h2, w3b, preferred_element_type=jnp.float32) \
            + b3_ref[0]
        w_col = jnp.sum(wts_ref[sl, :] * onehot, axis=-1, keepdims=True)
        contrib = w_col * o

        @pl.when(e == 0)
        def _init():
            out_ref[sl, :] = contrib

        @pl.when(e != 0)
        def _acc():
            out_ref[sl, :] += contrib


def kernel(x, gate_W, gate_b, W1, b1, W2, b2, W3, b3):
    n, d = x.shape
    grid = (n // _TB, _N_EXPERTS)
    return pl.pallas_call(
        _moe_body,
        grid=grid,
        in_specs=[
            pl.BlockSpec((_TB, d), lambda t, e: (t, 0)),
            pl.BlockSpec((d, _N_EXPERTS), lambda t, e: (0, 0)),
            pl.BlockSpec((1, _N_EXPERTS), lambda t, e: (0, 0)),
            pl.BlockSpec((1, d, W1.shape[2]), lambda t, e: (e, 0, 0)),
            pl.BlockSpec((1, 1, b1.shape[1]), lambda t, e: (e, 0, 0)),
            pl.BlockSpec((1, W2.shape[1], W2.shape[2]), lambda t, e: (e, 0, 0)),
            pl.BlockSpec((1, 1, b2.shape[1]), lambda t, e: (e, 0, 0)),
            pl.BlockSpec((1, W3.shape[1], W3.shape[2]), lambda t, e: (e, 0, 0)),
            pl.BlockSpec((1, 1, b3.shape[1]), lambda t, e: (e, 0, 0)),
        ],
        out_specs=pl.BlockSpec((_TB, W3.shape[2]), lambda t, e: (t, 0)),
        out_shape=jax.ShapeDtypeStruct((n, W3.shape[2]), jnp.float32),
        scratch_shapes=[pltpu.VMEM((_TB, _N_EXPERTS), jnp.float32)],
        compiler_params=pltpu.CompilerParams(
            dimension_semantics=("parallel", "arbitrary"),
            vmem_limit_bytes=100 * 1024 * 1024,
        ),
    )(x, gate_W, gate_b.reshape(1, -1), W1, b1[:, None, :], W2,
      b2[:, None, :], W3, b3[:, None, :])


# speedup vs baseline: 1.0406x; 1.0406x over previous
import jax
import jax.numpy as jnp
import numpy as np
from jax.experimental import pallas as pl
from jax.experimental.pallas import tpu as pltpu

_N_EXPERTS = 8
_N_ACTIVE = 5
_TEMP = float(np.e)
_TB = 2048
_SUB = 256


def _moe_body(x_ref, gw_ref, gb_ref, w1_ref, b1_ref, w2_ref, b2_ref,
              w3_ref, b3_ref, out_ref, wts_ref, xb_ref):
    e = pl.program_id(0)

    @pl.when(e == 0)
    def _gate():
        xb_ref[...] = x_ref[...].astype(jnp.bfloat16)
        st = (jax.lax.dot_general(
            gw_ref[...], x_ref[...], (((0,), (1,)), ((), ())),
            preferred_element_type=jnp.float32)
            + gb_ref[...].reshape(_N_EXPERTS, 1)) / _TEMP
        m = jnp.max(st, axis=0, keepdims=True)
        ex = jnp.exp(st - m)
        probs = ex / jnp.sum(ex, axis=0, keepdims=True)
        idx = jax.lax.broadcasted_iota(jnp.int32, probs.shape, 0)
        rows = []
        for i in range(_N_EXPERTS):
            pi = probs[i:i + 1, :]
            beats = (probs > pi).astype(jnp.float32) + \
                jnp.where(probs == pi, (idx < i).astype(jnp.float32), 0.0)
            rank = jnp.sum(beats, axis=0, keepdims=True)
            rows.append((rank < _N_ACTIVE).astype(jnp.float32))
        mask = jnp.concatenate(rows, axis=0)
        w = probs * mask
        w = w / (jnp.sum(w, axis=0, keepdims=True) + 1e-8)
        wts_ref[...] = jnp.transpose(w)

    w1b = w1_ref[0].astype(jnp.bfloat16)
    w2b = w2_ref[0].astype(jnp.bfloat16)
    w3b = w3_ref[0].astype(jnp.bfloat16)
    onehot = (jax.lax.broadcasted_iota(jnp.int32, (_SUB, _N_EXPERTS), 1)
              == e).astype(jnp.float32)

    for s in range(_TB // _SUB):
        sl = pl.ds(s * _SUB, _SUB)
        xs = xb_ref[sl, :]
        h1 = jnp.maximum(
            jnp.dot(xs, w1b, preferred_element_type=jnp.float32)
            + b1_ref[0], 0.0).astype(jnp.bfloat16)
        h2 = jnp.maximum(
            jnp.dot(h1, w2b, preferred_element_type=jnp.float32)
            + b2_ref[0], 0.0).astype(jnp.bfloat16)
        o = jnp.dot(h2, w3b, preferred_element_type=jnp.float32) \
            + b3_ref[0]
        w_col = jnp.sum(wts_ref[sl, :] * onehot, axis=-1, keepdims=True)
        contrib = w_col * o

        @pl.when(e == 0)
        def _init():
            out_ref[sl, :] = contrib

        @pl.when(e != 0)
        def _acc():
            out_ref[sl, :] += contrib


def kernel(x, gate_W, gate_b, W1, b1, W2, b2, W3, b3):
    n, d = x.shape
    return pl.pallas_call(
        _moe_body,
        grid=(_N_EXPERTS,),
        in_specs=[
            pl.BlockSpec((_TB, d), lambda e: (0, 0)),
            pl.BlockSpec((d, _N_EXPERTS), lambda e: (0, 0)),
            pl.BlockSpec((1, _N_EXPERTS), lambda e: (0, 0)),
            pl.BlockSpec((1, d, W1.shape[2]), lambda e: (e, 0, 0)),
            pl.BlockSpec((1, 1, b1.shape[1]), lambda e: (e, 0, 0)),
            pl.BlockSpec((1, W2.shape[1], W2.shape[2]), lambda e: (e, 0, 0)),
            pl.BlockSpec((1, 1, b2.shape[1]), lambda e: (e, 0, 0)),
            pl.BlockSpec((1, W3.shape[1], W3.shape[2]), lambda e: (e, 0, 0)),
            pl.BlockSpec((1, 1, b3.shape[1]), lambda e: (e, 0, 0)),
        ],
        out_specs=pl.BlockSpec((_TB, W3.shape[2]), lambda e: (0, 0)),
        out_shape=jax.ShapeDtypeStruct((n, W3.shape[2]), jnp.float32),
        scratch_shapes=[
            pltpu.VMEM((_TB, _N_EXPERTS), jnp.float32),
            pltpu.VMEM((_TB, d), jnp.bfloat16),
        ],
        compiler_params=pltpu.CompilerParams(
            dimension_semantics=("arbitrary",),
            vmem_limit_bytes=100 * 1024 * 1024,
        ),
    )(x, gate_W, gate_b.reshape(1, -1), W1, b1[:, None, :], W2,
      b2[:, None, :], W3, b3[:, None, :])


# speedup vs baseline: 1.1454x; 1.1007x over previous
import jax
import jax.numpy as jnp
import numpy as np
from jax.experimental import pallas as pl
from jax.experimental.pallas import tpu as pltpu

_N_EXPERTS = 8
_N_ACTIVE = 5
_TEMP = float(np.e)
_TB = 2048
_SUB = 2048


def _moe_body(x_ref, gw_ref, gb_ref, w1_ref, b1_ref, w2_ref, b2_ref,
              w3_ref, b3_ref, out_ref, wts_ref, xb_ref):
    e = pl.program_id(0)

    @pl.when(e == 0)
    def _gate():
        xb_ref[...] = x_ref[...].astype(jnp.bfloat16)
        st = (jax.lax.dot_general(
            gw_ref[...], x_ref[...], (((0,), (1,)), ((), ())),
            preferred_element_type=jnp.float32)
            + gb_ref[...].reshape(_N_EXPERTS, 1)) / _TEMP
        m = jnp.max(st, axis=0, keepdims=True)
        ex = jnp.exp(st - m)
        probs = ex / jnp.sum(ex, axis=0, keepdims=True)
        idx = jax.lax.broadcasted_iota(jnp.int32, probs.shape, 0)
        rows = []
        for i in range(_N_EXPERTS):
            pi = probs[i:i + 1, :]
            beats = (probs > pi).astype(jnp.float32) + \
                jnp.where(probs == pi, (idx < i).astype(jnp.float32), 0.0)
            rank = jnp.sum(beats, axis=0, keepdims=True)
            rows.append((rank < _N_ACTIVE).astype(jnp.float32))
        mask = jnp.concatenate(rows, axis=0)
        w = probs * mask
        w = w / (jnp.sum(w, axis=0, keepdims=True) + 1e-8)
        wts_ref[...] = jnp.transpose(w)

    w1b = w1_ref[0].astype(jnp.bfloat16)
    w2b = w2_ref[0].astype(jnp.bfloat16)
    w3b = w3_ref[0].astype(jnp.bfloat16)
    onehot = (jax.lax.broadcasted_iota(jnp.int32, (_SUB, _N_EXPERTS), 1)
              == e).astype(jnp.float32)

    for s in range(_TB // _SUB):
        sl = pl.ds(s * _SUB, _SUB)
        xs = xb_ref[sl, :]
        h1 = jnp.maximum(
            jnp.dot(xs, w1b, preferred_element_type=jnp.float32)
            + b1_ref[0], 0.0).astype(jnp.bfloat16)
        h2 = jnp.maximum(
            jnp.dot(h1, w2b, preferred_element_type=jnp.float32)
            + b2_ref[0], 0.0).astype(jnp.bfloat16)
        o = jnp.dot(h2, w3b, preferred_element_type=jnp.float32) \
            + b3_ref[0]
        w_col = jnp.sum(wts_ref[sl, :] * onehot, axis=-1, keepdims=True)
        contrib = w_col * o

        @pl.when(e == 0)
        def _init():
            out_ref[sl, :] = contrib

        @pl.when(e != 0)
        def _acc():
            out_ref[sl, :] += contrib


def kernel(x, gate_W, gate_b, W1, b1, W2, b2, W3, b3):
    n, d = x.shape
    return pl.pallas_call(
        _moe_body,
        grid=(_N_EXPERTS,),
        in_specs=[
            pl.BlockSpec((_TB, d), lambda e: (0, 0)),
            pl.BlockSpec((d, _N_EXPERTS), lambda e: (0, 0)),
            pl.BlockSpec((1, _N_EXPERTS), lambda e: (0, 0)),
            pl.BlockSpec((1, d, W1.shape[2]), lambda e: (e, 0, 0)),
            pl.BlockSpec((1, 1, b1.shape[1]), lambda e: (e, 0, 0)),
            pl.BlockSpec((1, W2.shape[1], W2.shape[2]), lambda e: (e, 0, 0)),
            pl.BlockSpec((1, 1, b2.shape[1]), lambda e: (e, 0, 0)),
            pl.BlockSpec((1, W3.shape[1], W3.shape[2]), lambda e: (e, 0, 0)),
            pl.BlockSpec((1, 1, b3.shape[1]), lambda e: (e, 0, 0)),
        ],
        out_specs=pl.BlockSpec((_TB, W3.shape[2]), lambda e: (0, 0)),
        out_shape=jax.ShapeDtypeStruct((n, W3.shape[2]), jnp.float32),
        scratch_shapes=[
            pltpu.VMEM((_TB, _N_EXPERTS), jnp.float32),
            pltpu.VMEM((_TB, d), jnp.bfloat16),
        ],
        compiler_params=pltpu.CompilerParams(
            dimension_semantics=("arbitrary",),
            vmem_limit_bytes=100 * 1024 * 1024,
        ),
    )(x, gate_W, gate_b.reshape(1, -1), W1, b1[:, None, :], W2,
      b2[:, None, :], W3, b3[:, None, :])
